# baseline (speedup 1.0000x reference)
import functools

import jax
import jax.numpy as jnp
from jax import lax
from jax.experimental import pallas as pl
from jax.experimental.pallas import tpu as pltpu
from jax.experimental.pallas import tpu_sc as plsc

EMBED = 32
NC = 2
NS = 16
NW = NC * NS
CHUNK = 128


def _make_kernel(batch):
    bpw = batch // NW
    nch = bpw // CHUNK
    out_dim = 4 * EMBED

    mesh = plsc.VectorSubcoreMesh(core_axis_name="c", subcore_axis_name="s")

    @functools.partial(
        pl.kernel,
        out_type=jax.ShapeDtypeStruct((batch, out_dim), jnp.float32),
        mesh=mesh,
        scratch_types=(
            [pltpu.VMEM((nch, CHUNK), jnp.int32) for _ in range(4)]
            + [pltpu.VMEM((bpw, EMBED), jnp.float32) for _ in range(4)]
            + [pltpu.SemaphoreType.DMA]
        ),
        compiler_params=pltpu.CompilerParams(use_tc_tiling_on_sc=False),
    )
    def gather_kernel(u_t, r_t, i_t, n_t, uix, rix, iix, nix, out,
                      vu, vr, vi, vn, gu, gr, gi, gn, sem):
        wid = lax.axis_index("s") * NC + lax.axis_index("c")
        base = wid * bpw
        tables = (u_t, r_t, i_t, n_t)
        idx_hbm = (uix, rix, iix, nix)
        idx_v = (vu, vr, vi, vn)
        rows_v = (gu, gr, gi, gn)
        for t in range(4):
            pltpu.sync_copy(idx_hbm[t].at[wid], idx_v[t])
        copies = []
        for t in range(4):
            for j in range(nch):
                copies.append(pltpu.async_copy(
                    tables[t].at[idx_v[t].at[j]],
                    rows_v[t].at[pl.ds(j * CHUNK, CHUNK)],
                    sem))
        for cp in copies:
            cp.wait()
        for t in range(4):
            pltpu.sync_copy(
                rows_v[t],
                out.at[pl.ds(base, bpw), pl.ds(t * EMBED, EMBED)])

    return gather_kernel


def kernel(uid, rid, ing, nut, user_table, recipe_table, ingredient_table,
           nutrition_table):
    batch = uid.shape[0]
    bpw = batch // NW
    nch = bpw // CHUNK
    uix = uid.astype(jnp.int32).reshape(NW, nch, CHUNK)
    rix = rid.astype(jnp.int32).reshape(NW, nch, CHUNK)
    iix = ing.astype(jnp.int32).reshape(NW, nch, CHUNK)
    nix = nut.astype(jnp.int32).reshape(NW, nch, CHUNK)
    fn = _make_kernel(batch)
    return fn(user_table, recipe_table, ingredient_table, nutrition_table,
              uix, rix, iix, nix)
